# Initial kernel scaffold; baseline (speedup 1.0000x reference)
import jax
import jax.numpy as jnp
from jax.experimental import pallas as pl


def kernel(x, edge_index, edge_attr, W0, b0, Watt0, batt0, W1, b1, Watt1, batt1):
    raise NotImplementedError("write your pallas kernel here")


# speedup vs baseline: 3.7960x; 3.7960x over previous
import functools

import jax
import jax.numpy as jnp
from jax import lax
from jax.experimental import pallas as pl
from jax.experimental.pallas import tpu as pltpu
from jax.experimental.pallas import tpu_sc as plsc

N = 10000
E = 320000
D = 128
DE = 16


_MAXN = 1.0 - 1e-5


def _artanh(x):
    x = jnp.clip(x, -1.0 + 1e-7, 1.0 - 1e-7)
    return 0.5 * jnp.log((1.0 + x) / (1.0 - x))


def _rownorm(x):
    return jnp.sqrt(jnp.sum(x * x, axis=-1, keepdims=True))


def _proj(x):
    n = jnp.maximum(_rownorm(x), 1e-15)
    return jnp.where(n > _MAXN, x / n * _MAXN, x)


def _expmap0(u):
    n = jnp.maximum(_rownorm(u), 1e-15)
    return jnp.tanh(n) * u / n


def _logmap0(p):
    n = jnp.maximum(_rownorm(p), 1e-15)
    return _artanh(n) * p / n


def _dense_pre(xh, wt, b):
    xn = jnp.maximum(_rownorm(xh), 1e-15)
    mx = jnp.dot(xh, wt, preferred_element_type=jnp.float32)
    mxn = jnp.maximum(_rownorm(mx), 1e-15)
    mv = jnp.tanh(mxn / xn * _artanh(xn)) * mx / mxn
    mv = _proj(mv)
    hb = _proj(_expmap0(b))
    x2 = jnp.sum(mv * mv, axis=-1, keepdims=True)
    y2 = jnp.sum(hb * hb, axis=-1, keepdims=True)
    xy = jnp.sum(mv * hb, axis=-1, keepdims=True)
    num = (1.0 + 2.0 * xy + y2) * mv + (1.0 - x2) * hb
    den = 1.0 + 2.0 * xy + x2 * y2
    h = _proj(num / jnp.maximum(den, 1e-15))
    return _logmap0(h)


def _dense_post(s):
    h = _proj(_expmap0(s))
    xt = jnp.tanh(_logmap0(h))
    return _proj(_expmap0(xt))


_BLK = 2000
_EBLK = 8000


def _edge_body(ea_ref, we0_ref, we1_ref, b0_ref, b1_ref, ae0_ref, ae1_ref):
    ea = ea_ref[...]
    et0 = _logmap0(_proj(_expmap0(ea)))
    ae0_ref[...] = (jnp.sum(et0 * we0_ref[...], axis=1, keepdims=True)
                    + b0_ref[0, 0])
    et1 = _logmap0(_proj(_expmap0(et0)))
    ae1_ref[...] = (jnp.sum(et1 * we1_ref[...], axis=1, keepdims=True)
                    + b1_ref[0, 0])


def _edge_call(edge_attr, we0, we1, b0, b1):
    grid = E // _EBLK
    full16 = pl.BlockSpec((1, DE), lambda i: (0, 0))
    one = pl.BlockSpec((1, 1), lambda i: (0, 0))
    return pl.pallas_call(
        _edge_body,
        grid=(grid,),
        in_specs=[pl.BlockSpec((_EBLK, DE), lambda i: (i, 0)),
                  full16, full16, one, one],
        out_specs=[pl.BlockSpec((_EBLK, 1), lambda i: (i, 0)),
                   pl.BlockSpec((_EBLK, 1), lambda i: (i, 0))],
        out_shape=[jax.ShapeDtypeStruct((E, 1), jnp.float32),
                   jax.ShapeDtypeStruct((E, 1), jnp.float32)],
    )(edge_attr, we0, we1, b0, b1)


def _pre_tail(xt, ws_ref, wd_ref, xt_ref, as_ref, ad_ref):
    xt_ref[...] = xt
    as_ref[...] = jnp.sum(xt * ws_ref[...], axis=1, keepdims=True)
    ad_ref[...] = jnp.sum(xt * wd_ref[...], axis=1, keepdims=True)


def _pre0_body(x_ref, wt_ref, b_ref, ws_ref, wd_ref, xt_ref, as_ref, ad_ref):
    xh = _proj(_expmap0(x_ref[...]))
    xt = _dense_pre(xh, wt_ref[...], b_ref[...])
    _pre_tail(xt, ws_ref, wd_ref, xt_ref, as_ref, ad_ref)


def _mid_body(sa_ref, sb_ref, wt_ref, b_ref, ws_ref, wd_ref,
              xt_ref, as_ref, ad_ref):
    xh = _dense_post(sa_ref[...] + sb_ref[...])
    xt = _dense_pre(xh, wt_ref[...], b_ref[...])
    _pre_tail(xt, ws_ref, wd_ref, xt_ref, as_ref, ad_ref)


def _post_body(sa_ref, sb_ref, o_ref):
    o_ref[...] = _dense_post(sa_ref[...] + sb_ref[...])


_rows = pl.BlockSpec((_BLK, D), lambda i: (i, 0))
_full = pl.BlockSpec((D, D), lambda i: (0, 0))
_vec = pl.BlockSpec((1, D), lambda i: (0, 0))
_col = pl.BlockSpec((_BLK, 1), lambda i: (i, 0))
_node_outs = [jax.ShapeDtypeStruct((N, D), jnp.float32),
              jax.ShapeDtypeStruct((N, 1), jnp.float32),
              jax.ShapeDtypeStruct((N, 1), jnp.float32)]


def _pre0_call(x, wt, b, ws, wd):
    return pl.pallas_call(
        _pre0_body, grid=(N // _BLK,),
        in_specs=[_rows, _full, _vec, _vec, _vec],
        out_specs=[_rows, _col, _col],
        out_shape=_node_outs,
    )(x, wt, b, ws, wd)


def _mid_call(sa, sb, wt, b, ws, wd):
    return pl.pallas_call(
        _mid_body, grid=(N // _BLK,),
        in_specs=[_rows, _rows, _full, _vec, _vec, _vec],
        out_specs=[_rows, _col, _col],
        out_shape=_node_outs,
    )(sa, sb, wt, b, ws, wd)


def _post_call(sa, sb):
    return pl.pallas_call(
        _post_body, grid=(N // _BLK,),
        in_specs=[_rows, _rows],
        out_specs=_rows,
        out_shape=jax.ShapeDtypeStruct((N, D), jnp.float32),
    )(sa, sb)


_NC = 2
_NS = 16
_NW = _NC * _NS
_B = 80
_EPT = E // _NW
_NBLK = _EPT // _B
_NP = 10240
_RPT = _NP // _NS
_ZR = 128

_sc_mesh = plsc.VectorSubcoreMesh(core_axis_name="c", subcore_axis_name="s")


@functools.partial(
    pl.kernel, mesh=_sc_mesh,
    out_type=jax.ShapeDtypeStruct((_NC, _NP, D), jnp.float32),
    compiler_params=pltpu.CompilerParams(needs_layout_passes=False),
    scratch_types=[
        pltpu.VMEM((N,), jnp.float32),
        pltpu.VMEM((N,), jnp.float32),
        pltpu.VMEM((_B,), jnp.int32),
        pltpu.VMEM((1, _B), jnp.int32),
        pltpu.VMEM((_B,), jnp.float32),
        pltpu.VMEM((_B,), jnp.float32),
        pltpu.VMEM((_B, D), jnp.float32),
        pltpu.VMEM((_ZR, D), jnp.float32),
        pltpu.VMEM_SHARED((_NP, D), jnp.float32),
        pltpu.SemaphoreType.DMA,
    ],
)
def _sc_agg(xt_hbm, src_hbm, dst_hbm, ae_hbm, asrc_hbm, adst_hbm, out_hbm,
            asrc_v, adst_v, src_v, dst_v, ae_v, att_v, rows_v, zbuf, acc,
            sem):
    c = lax.axis_index("c")
    s = lax.axis_index("s")
    wid = c * _NS + s

    def _zrow(i, carry):
        for k in range(D // 16):
            zbuf[i, pl.ds(k * 16, 16)] = jnp.zeros((16,), jnp.float32)
        return carry

    lax.fori_loop(0, _ZR, _zrow, 0)
    for k in range(_RPT // _ZR):
        pltpu.sync_copy(zbuf, acc.at[pl.ds(s * _RPT + k * _ZR, _ZR)])
    plsc.subcore_barrier()

    pltpu.sync_copy(asrc_hbm, asrc_v)
    pltpu.sync_copy(adst_hbm, adst_v)

    base = wid * _EPT

    def _block(j, carry):
        off = base + j * _B
        pltpu.sync_copy(src_hbm.at[pl.ds(off, _B)], src_v)
        pltpu.sync_copy(dst_hbm.at[pl.ds(off, _B)], dst_v.at[0])
        pltpu.sync_copy(ae_hbm.at[pl.ds(off, _B)], ae_v)
        gat = pltpu.async_copy(xt_hbm.at[src_v], rows_v, sem)
        for jj in range(_B // 16):
            sl = pl.ds(jj * 16, 16)
            a_s = plsc.load_gather(asrc_v, [src_v[sl]])
            a_d = plsc.load_gather(adst_v, [dst_v[0, sl]])
            z = a_s + a_d + ae_v[sl]
            att_v[sl] = 1.0 / (1.0 + jnp.exp(-z))
        gat.wait()

        def _egrp(g, carry2):
            att16 = att_v[pl.ds(g * 16, 16)]
            for l in range(16):
                a = att16[l]
                i = g * 16 + l
                for k in range(D // 16):
                    sl = pl.ds(k * 16, 16)
                    rows_v[i, sl] = rows_v[i, sl] * a
            return carry2

        lax.fori_loop(0, _B // 16, _egrp, 0)
        pltpu.sync_copy(rows_v, acc.at[dst_v.at[0]], add=True)
        return carry

    lax.fori_loop(0, _NBLK, _block, 0)
    plsc.subcore_barrier()

    for k in range(_RPT // _ZR):
        rs = s * _RPT + k * _ZR
        pltpu.sync_copy(acc.at[pl.ds(rs, _ZR)], zbuf)
        pltpu.sync_copy(zbuf, out_hbm.at[c, pl.ds(rs, _ZR)])


def kernel(x, edge_index, edge_attr, W0, b0, Watt0, batt0, W1, b1, Watt1,
           batt1):
    src = edge_index[0]
    dst = edge_index[1]
    ws0 = Watt0[:D, 0].reshape(1, D)
    wd0 = Watt0[D:2 * D, 0].reshape(1, D)
    we0 = Watt0[2 * D:, 0].reshape(1, DE)
    ws1 = Watt1[:D, 0].reshape(1, D)
    wd1 = Watt1[D:2 * D, 0].reshape(1, D)
    we1 = Watt1[2 * D:, 0].reshape(1, DE)

    ae0, ae1 = _edge_call(edge_attr, we0, we1,
                          batt0.reshape(1, 1), batt1.reshape(1, 1))
    ae0 = ae0.reshape(E)
    ae1 = ae1.reshape(E)

    xt0, as0, ad0 = _pre0_call(x, W0.T, b0.reshape(1, D), ws0, wd0)
    part = _sc_agg(xt0, src, dst, ae0, as0.reshape(N), ad0.reshape(N))
    xt1, as1, ad1 = _mid_call(part[0, :N], part[1, :N], W1.T,
                              b1.reshape(1, D), ws1, wd1)
    part = _sc_agg(xt1, src, dst, ae1, as1.reshape(N), ad1.reshape(N))
    return _post_call(part[0, :N], part[1, :N])


# speedup vs baseline: 4.8296x; 1.2723x over previous
import functools

import jax
import jax.numpy as jnp
from jax import lax
from jax.experimental import pallas as pl
from jax.experimental.pallas import tpu as pltpu
from jax.experimental.pallas import tpu_sc as plsc

N = 10000
E = 320000
D = 128
DE = 16


_MAXN = 1.0 - 1e-5


def _artanh(x):
    x = jnp.clip(x, -1.0 + 1e-7, 1.0 - 1e-7)
    return 0.5 * jnp.log((1.0 + x) / (1.0 - x))


def _rownorm(x):
    return jnp.sqrt(jnp.sum(x * x, axis=-1, keepdims=True))


def _proj(x):
    n = jnp.maximum(_rownorm(x), 1e-15)
    return jnp.where(n > _MAXN, x / n * _MAXN, x)


def _expmap0(u):
    n = jnp.maximum(_rownorm(u), 1e-15)
    return jnp.tanh(n) * u / n


def _logmap0(p):
    n = jnp.maximum(_rownorm(p), 1e-15)
    return _artanh(n) * p / n


def _dense_pre(xh, wt, b):
    xn = jnp.maximum(_rownorm(xh), 1e-15)
    mx = jnp.dot(xh, wt, preferred_element_type=jnp.float32)
    mxn = jnp.maximum(_rownorm(mx), 1e-15)
    mv = jnp.tanh(mxn / xn * _artanh(xn)) * mx / mxn
    mv = _proj(mv)
    hb = _proj(_expmap0(b))
    x2 = jnp.sum(mv * mv, axis=-1, keepdims=True)
    y2 = jnp.sum(hb * hb, axis=-1, keepdims=True)
    xy = jnp.sum(mv * hb, axis=-1, keepdims=True)
    num = (1.0 + 2.0 * xy + y2) * mv + (1.0 - x2) * hb
    den = 1.0 + 2.0 * xy + x2 * y2
    h = _proj(num / jnp.maximum(den, 1e-15))
    return _logmap0(h)


def _dense_post(s):
    h = _proj(_expmap0(s))
    xt = jnp.tanh(_logmap0(h))
    return _proj(_expmap0(xt))


_BLK = 2000
_EBLK = 8000


def _edge_body(ea_ref, we0_ref, we1_ref, b0_ref, b1_ref, ae0_ref, ae1_ref):
    ea = ea_ref[...]
    et0 = _logmap0(_proj(_expmap0(ea)))
    ae0_ref[...] = (jnp.sum(et0 * we0_ref[...], axis=1, keepdims=True)
                    + b0_ref[0, 0])
    et1 = _logmap0(_proj(_expmap0(et0)))
    ae1_ref[...] = (jnp.sum(et1 * we1_ref[...], axis=1, keepdims=True)
                    + b1_ref[0, 0])


def _edge_call(edge_attr, we0, we1, b0, b1):
    grid = E // _EBLK
    full16 = pl.BlockSpec((1, DE), lambda i: (0, 0))
    one = pl.BlockSpec((1, 1), lambda i: (0, 0))
    return pl.pallas_call(
        _edge_body,
        grid=(grid,),
        in_specs=[pl.BlockSpec((_EBLK, DE), lambda i: (i, 0)),
                  full16, full16, one, one],
        out_specs=[pl.BlockSpec((_EBLK, 1), lambda i: (i, 0)),
                   pl.BlockSpec((_EBLK, 1), lambda i: (i, 0))],
        out_shape=[jax.ShapeDtypeStruct((E, 1), jnp.float32),
                   jax.ShapeDtypeStruct((E, 1), jnp.float32)],
    )(edge_attr, we0, we1, b0, b1)


def _pre_tail(xt, ws_ref, wd_ref, xt_ref, as_ref, ad_ref):
    xt_ref[...] = xt
    as_ref[...] = jnp.sum(xt * ws_ref[...], axis=1, keepdims=True)
    ad_ref[...] = jnp.sum(xt * wd_ref[...], axis=1, keepdims=True)


def _pre0_body(x_ref, wt_ref, b_ref, ws_ref, wd_ref, xt_ref, as_ref, ad_ref):
    xh = _proj(_expmap0(x_ref[...]))
    xt = _dense_pre(xh, wt_ref[...], b_ref[...])
    _pre_tail(xt, ws_ref, wd_ref, xt_ref, as_ref, ad_ref)


def _mid_body(sa_ref, sb_ref, wt_ref, b_ref, ws_ref, wd_ref,
              xt_ref, as_ref, ad_ref):
    xh = _dense_post(sa_ref[...] + sb_ref[...])
    xt = _dense_pre(xh, wt_ref[...], b_ref[...])
    _pre_tail(xt, ws_ref, wd_ref, xt_ref, as_ref, ad_ref)


def _post_body(sa_ref, sb_ref, o_ref):
    o_ref[...] = _dense_post(sa_ref[...] + sb_ref[...])


_rows = pl.BlockSpec((_BLK, D), lambda i: (i, 0))
_full = pl.BlockSpec((D, D), lambda i: (0, 0))
_vec = pl.BlockSpec((1, D), lambda i: (0, 0))
_col = pl.BlockSpec((_BLK, 1), lambda i: (i, 0))
_node_outs = [jax.ShapeDtypeStruct((N, D), jnp.float32),
              jax.ShapeDtypeStruct((N, 1), jnp.float32),
              jax.ShapeDtypeStruct((N, 1), jnp.float32)]


def _pre0_call(x, wt, b, ws, wd):
    return pl.pallas_call(
        _pre0_body, grid=(N // _BLK,),
        in_specs=[_rows, _full, _vec, _vec, _vec],
        out_specs=[_rows, _col, _col],
        out_shape=_node_outs,
    )(x, wt, b, ws, wd)


def _mid_call(sa, sb, wt, b, ws, wd):
    return pl.pallas_call(
        _mid_body, grid=(N // _BLK,),
        in_specs=[_rows, _rows, _full, _vec, _vec, _vec],
        out_specs=[_rows, _col, _col],
        out_shape=_node_outs,
    )(sa, sb, wt, b, ws, wd)


def _post_call(sa, sb):
    return pl.pallas_call(
        _post_body, grid=(N // _BLK,),
        in_specs=[_rows, _rows],
        out_specs=_rows,
        out_shape=jax.ShapeDtypeStruct((N, D), jnp.float32),
    )(sa, sb)


_NC = 2
_NS = 16
_NW = _NC * _NS
_B = 80
_EPT = E // _NW
_NBLK = _EPT // _B
_NP = 10240
_RPT = _NP // _NS
_ZR = 32

_CHB = 5
_NCH = _NBLK // _CHB
_NSLOT = 2

_sc_mesh = plsc.VectorSubcoreMesh(core_axis_name="c", subcore_axis_name="s")


@functools.partial(
    pl.kernel, mesh=_sc_mesh,
    out_type=jax.ShapeDtypeStruct((_NC, _NP, D), jnp.float32),
    compiler_params=pltpu.CompilerParams(needs_layout_passes=False),
    scratch_types=[
        pltpu.VMEM((N,), jnp.float32),
        pltpu.VMEM((N,), jnp.float32),
        pltpu.VMEM((_CHB, _B), jnp.int32),
        pltpu.VMEM((_CHB, _B), jnp.int32),
        pltpu.VMEM((_CHB, _B), jnp.float32),
        pltpu.VMEM((_NSLOT, _B, D), jnp.float32),
        pltpu.VMEM((_ZR, D), jnp.float32),
        pltpu.VMEM_SHARED((_NP, D), jnp.float32),
        pltpu.SemaphoreType.DMA((_NSLOT,)),
        pltpu.SemaphoreType.DMA((_NSLOT,)),
    ],
)
def _sc_agg(xt_hbm, src_hbm, dst_hbm, ae_hbm, asrc_hbm, adst_hbm, out_hbm,
            asrc_v, adst_v, srcc, dstc, aec, rows, zbuf, acc, gsem, ssem):
    c = lax.axis_index("c")
    s = lax.axis_index("s")
    wid = c * _NS + s

    def _zrow(i, carry):
        for k in range(D // 16):
            zbuf[i, pl.ds(k * 16, 16)] = jnp.zeros((16,), jnp.float32)
        return carry

    lax.fori_loop(0, _ZR, _zrow, 0)
    for k in range(_RPT // _ZR):
        pltpu.sync_copy(zbuf, acc.at[pl.ds(s * _RPT + k * _ZR, _ZR)])
    plsc.subcore_barrier()

    pltpu.sync_copy(asrc_hbm, asrc_v)
    pltpu.sync_copy(adst_hbm, adst_v)

    def _chunk(ch, carry):
        pltpu.sync_copy(src_hbm.at[wid, ch], srcc)
        pltpu.sync_copy(dst_hbm.at[wid, ch], dstc)
        pltpu.sync_copy(ae_hbm.at[wid, ch], aec)
        gats = [pltpu.async_copy(xt_hbm.at[srcc.at[b]], rows.at[b],
                                 gsem.at[b]) for b in range(_NSLOT)]
        scs = [None] * _NSLOT
        for b in range(_CHB):
            sl_ = b % _NSLOT
            gats[sl_].wait()

            def _grp(g, carry2, b=b, sl_=sl_):
                gsl = pl.ds(g * 16, 16)
                a_s = plsc.load_gather(asrc_v, [srcc[b, gsl]])
                a_d = plsc.load_gather(adst_v, [dstc[b, gsl]])
                att16 = 1.0 / (1.0 + jnp.exp(-(a_s + a_d + aec[b, gsl])))
                for l in range(16):
                    a = att16[l]
                    i = g * 16 + l
                    for k in range(D // 16):
                        ksl = pl.ds(k * 16, 16)
                        rows[sl_, i, ksl] = rows[sl_, i, ksl] * a
                return carry2

            lax.fori_loop(0, _B // 16, _grp, 0)
            scs[sl_] = pltpu.async_copy(
                rows.at[sl_], acc.at[dstc.at[b]], ssem.at[sl_], add=True)
            if b + _NSLOT < _CHB:
                scs[sl_].wait()
                gats[sl_] = pltpu.async_copy(
                    xt_hbm.at[srcc.at[b + _NSLOT]], rows.at[sl_],
                    gsem.at[sl_])
        for h in scs:
            h.wait()
        return carry

    lax.fori_loop(0, _NCH, _chunk, 0)
    plsc.subcore_barrier()

    for k in range(_RPT // _ZR):
        rs = s * _RPT + k * _ZR
        pltpu.sync_copy(acc.at[pl.ds(rs, _ZR)], zbuf)
        pltpu.sync_copy(zbuf, out_hbm.at[c, pl.ds(rs, _ZR)])


def kernel(x, edge_index, edge_attr, W0, b0, Watt0, batt0, W1, b1, Watt1,
           batt1):
    src = edge_index[0].reshape(_NW, _NCH, _CHB, _B)
    dst = edge_index[1].reshape(_NW, _NCH, _CHB, _B)
    ws0 = Watt0[:D, 0].reshape(1, D)
    wd0 = Watt0[D:2 * D, 0].reshape(1, D)
    we0 = Watt0[2 * D:, 0].reshape(1, DE)
    ws1 = Watt1[:D, 0].reshape(1, D)
    wd1 = Watt1[D:2 * D, 0].reshape(1, D)
    we1 = Watt1[2 * D:, 0].reshape(1, DE)

    ae0, ae1 = _edge_call(edge_attr, we0, we1,
                          batt0.reshape(1, 1), batt1.reshape(1, 1))
    ae0 = ae0.reshape(_NW, _NCH, _CHB, _B)
    ae1 = ae1.reshape(_NW, _NCH, _CHB, _B)

    xt0, as0, ad0 = _pre0_call(x, W0.T, b0.reshape(1, D), ws0, wd0)
    part = _sc_agg(xt0, src, dst, ae0, as0.reshape(N), ad0.reshape(N))
    xt1, as1, ad1 = _mid_call(part[0, :N], part[1, :N], W1.T,
                              b1.reshape(1, D), ws1, wd1)
    part = _sc_agg(xt1, src, dst, ae1, as1.reshape(N), ad1.reshape(N))
    return _post_call(part[0, :N], part[1, :N])


# speedup vs baseline: 10.2733x; 2.1272x over previous
import functools

import jax
import jax.numpy as jnp
from jax import lax
from jax.experimental import pallas as pl
from jax.experimental.pallas import tpu as pltpu
from jax.experimental.pallas import tpu_sc as plsc

N = 10000
E = 320000
D = 128
DE = 16


_MAXN = 1.0 - 1e-5


def _artanh(x):
    x = jnp.clip(x, -1.0 + 1e-7, 1.0 - 1e-7)
    return 0.5 * jnp.log((1.0 + x) / (1.0 - x))


def _rownorm(x):
    return jnp.sqrt(jnp.sum(x * x, axis=-1, keepdims=True))


def _proj(x):
    n = jnp.maximum(_rownorm(x), 1e-15)
    return jnp.where(n > _MAXN, x / n * _MAXN, x)


def _expmap0(u):
    n = jnp.maximum(_rownorm(u), 1e-15)
    return jnp.tanh(n) * u / n


def _logmap0(p):
    n = jnp.maximum(_rownorm(p), 1e-15)
    return _artanh(n) * p / n


def _dense_pre(xh, wt, b):
    xn = jnp.maximum(_rownorm(xh), 1e-15)
    mx = jnp.dot(xh, wt, preferred_element_type=jnp.float32)
    mxn = jnp.maximum(_rownorm(mx), 1e-15)
    mv = jnp.tanh(mxn / xn * _artanh(xn)) * mx / mxn
    mv = _proj(mv)
    hb = _proj(_expmap0(b))
    x2 = jnp.sum(mv * mv, axis=-1, keepdims=True)
    y2 = jnp.sum(hb * hb, axis=-1, keepdims=True)
    xy = jnp.sum(mv * hb, axis=-1, keepdims=True)
    num = (1.0 + 2.0 * xy + y2) * mv + (1.0 - x2) * hb
    den = 1.0 + 2.0 * xy + x2 * y2
    h = _proj(num / jnp.maximum(den, 1e-15))
    return _logmap0(h)


def _dense_post(s):
    h = _proj(_expmap0(s))
    xt = jnp.tanh(_logmap0(h))
    return _proj(_expmap0(xt))


_BLK = 2000
_EBLK = 6400


def _hyp_scale(n):
    t = jnp.tanh(n)
    f = jnp.where(t > _MAXN, _MAXN / n, t / n)
    nh = jnp.minimum(t, _MAXN)
    return _artanh(nh) / jnp.maximum(nh, 1e-15) * f


def _edge_body(eat_ref, we0_ref, we1_ref, b0_ref, b1_ref, ae0_ref, ae1_ref):
    x = eat_ref[...]
    q = jnp.sum(x * x, axis=0, keepdims=True)
    na = jnp.maximum(jnp.sqrt(q), 1e-15)
    p0 = jnp.sum(x * we0_ref[...], axis=0, keepdims=True)
    p1 = jnp.sum(x * we1_ref[...], axis=0, keepdims=True)
    s0 = _hyp_scale(na)
    ae0_ref[...] = (s0 * p0 + b0_ref[0, 0]).reshape(1, 1, _EBLK)
    n1 = jnp.maximum(s0 * na, 1e-15)
    s1 = _hyp_scale(n1)
    ae1_ref[...] = (s1 * s0 * p1 + b1_ref[0, 0]).reshape(1, 1, _EBLK)


def _edge_call(eat, we0, we1, b0, b1):
    grid = E // _EBLK
    colw = pl.BlockSpec((DE, 1), lambda i: (0, 0))
    one = pl.BlockSpec((1, 1), lambda i: (0, 0))
    return pl.pallas_call(
        _edge_body,
        grid=(grid,),
        in_specs=[pl.BlockSpec((DE, _EBLK), lambda i: (0, i)),
                  colw, colw, one, one],
        out_specs=[pl.BlockSpec((1, 1, _EBLK), lambda i: (i, 0, 0)),
                   pl.BlockSpec((1, 1, _EBLK), lambda i: (i, 0, 0))],
        out_shape=[jax.ShapeDtypeStruct((E // _EBLK, 1, _EBLK), jnp.float32),
                   jax.ShapeDtypeStruct((E // _EBLK, 1, _EBLK), jnp.float32)],
    )(eat, we0, we1, b0, b1)


def _pre_tail(xt, ws_ref, wd_ref, xt_ref, as_ref, ad_ref):
    xt_ref[...] = xt
    as_ref[...] = jnp.sum(xt * ws_ref[...], axis=1, keepdims=True)
    ad_ref[...] = jnp.sum(xt * wd_ref[...], axis=1, keepdims=True)


def _pre0_body(x_ref, wt_ref, b_ref, ws_ref, wd_ref, xt_ref, as_ref, ad_ref):
    xh = _proj(_expmap0(x_ref[...]))
    xt = _dense_pre(xh, wt_ref[...], b_ref[...])
    _pre_tail(xt, ws_ref, wd_ref, xt_ref, as_ref, ad_ref)


def _mid_body(sa_ref, sb_ref, wt_ref, b_ref, ws_ref, wd_ref,
              xt_ref, as_ref, ad_ref):
    xh = _dense_post(sa_ref[...] + sb_ref[...])
    xt = _dense_pre(xh, wt_ref[...], b_ref[...])
    _pre_tail(xt, ws_ref, wd_ref, xt_ref, as_ref, ad_ref)


def _post_body(sa_ref, sb_ref, o_ref):
    o_ref[...] = _dense_post(sa_ref[...] + sb_ref[...])


_rows = pl.BlockSpec((_BLK, D), lambda i: (i, 0))
_full = pl.BlockSpec((D, D), lambda i: (0, 0))
_vec = pl.BlockSpec((1, D), lambda i: (0, 0))
_col = pl.BlockSpec((_BLK, 1), lambda i: (i, 0))
_node_outs = [jax.ShapeDtypeStruct((N, D), jnp.float32),
              jax.ShapeDtypeStruct((N, 1), jnp.float32),
              jax.ShapeDtypeStruct((N, 1), jnp.float32)]


def _pre0_call(x, wt, b, ws, wd):
    return pl.pallas_call(
        _pre0_body, grid=(N // _BLK,),
        in_specs=[_rows, _full, _vec, _vec, _vec],
        out_specs=[_rows, _col, _col],
        out_shape=_node_outs,
    )(x, wt, b, ws, wd)


def _mid_call(sa, sb, wt, b, ws, wd):
    return pl.pallas_call(
        _mid_body, grid=(N // _BLK,),
        in_specs=[_rows, _rows, _full, _vec, _vec, _vec],
        out_specs=[_rows, _col, _col],
        out_shape=_node_outs,
    )(sa, sb, wt, b, ws, wd)


def _post_call(sa, sb):
    return pl.pallas_call(
        _post_body, grid=(N // _BLK,),
        in_specs=[_rows, _rows],
        out_specs=_rows,
        out_shape=jax.ShapeDtypeStruct((N, D), jnp.float32),
    )(sa, sb)


_NC = 2
_NS = 16
_NW = _NC * _NS
_B = 80
_EPT = E // _NW
_NBLK = _EPT // _B
_NP = 10240
_RPT = _NP // _NS
_ZR = 32

_CHB = 5
_NCH = _NBLK // _CHB
_NSLOT = 2

_sc_mesh = plsc.VectorSubcoreMesh(core_axis_name="c", subcore_axis_name="s")


@functools.partial(
    pl.kernel, mesh=_sc_mesh,
    out_type=jax.ShapeDtypeStruct((_NC, _NP, D), jnp.float32),
    compiler_params=pltpu.CompilerParams(needs_layout_passes=False),
    scratch_types=[
        pltpu.VMEM((N,), jnp.float32),
        pltpu.VMEM((N,), jnp.float32),
        pltpu.VMEM((_CHB * _B,), jnp.int32),
        pltpu.VMEM((_CHB, _B), jnp.int32),
        pltpu.VMEM((_CHB * _B,), jnp.float32),
        pltpu.VMEM((_NSLOT, _B, D), jnp.float32),
        pltpu.VMEM((_ZR, D), jnp.float32),
        pltpu.VMEM_SHARED((_NP, D), jnp.float32),
        pltpu.SemaphoreType.DMA((_NSLOT,)),
        pltpu.SemaphoreType.DMA((_NSLOT,)),
    ],
)
def _sc_agg(xt_hbm, src_hbm, dst_hbm, ae_hbm, asrc_hbm, adst_hbm, out_hbm,
            asrc_v, adst_v, srcc, dstc, aec, rows, zbuf, acc, gsem, ssem):
    c = lax.axis_index("c")
    s = lax.axis_index("s")
    wid = c * _NS + s

    def _zrow(i, carry):
        for k in range(D // 16):
            zbuf[i, pl.ds(k * 16, 16)] = jnp.zeros((16,), jnp.float32)
        return carry

    lax.fori_loop(0, _ZR, _zrow, 0)
    for k in range(_RPT // _ZR):
        pltpu.sync_copy(zbuf, acc.at[pl.ds(s * _RPT + k * _ZR, _ZR)])
    plsc.subcore_barrier()

    pltpu.sync_copy(asrc_hbm, asrc_v)
    pltpu.sync_copy(adst_hbm, adst_v)

    def _chunk(ch, carry):
        off = wid * _EPT + ch * (_CHB * _B)
        pltpu.sync_copy(src_hbm.at[pl.ds(off, _CHB * _B)], srcc)
        pltpu.sync_copy(dst_hbm.at[wid, ch], dstc)
        pltpu.sync_copy(ae_hbm.at[pl.ds(off, _CHB * _B)], aec)
        gats = [pltpu.async_copy(xt_hbm.at[srcc.at[pl.ds(b * _B, _B)]],
                                 rows.at[b], gsem.at[b])
                for b in range(_NSLOT)]
        scs = [None] * _NSLOT
        for b in range(_CHB):
            sl_ = b % _NSLOT
            gats[sl_].wait()

            def _grp(g, carry2, b=b, sl_=sl_):
                gsl = pl.ds(b * _B + g * 16, 16)
                a_s = plsc.load_gather(asrc_v, [srcc[gsl]])
                a_d = plsc.load_gather(adst_v, [dstc[b, pl.ds(g * 16, 16)]])
                att16 = 1.0 / (1.0 + jnp.exp(-(a_s + a_d + aec[gsl])))
                for l in range(16):
                    a = att16[l]
                    i = g * 16 + l
                    for k in range(D // 16):
                        ksl = pl.ds(k * 16, 16)
                        rows[sl_, i, ksl] = rows[sl_, i, ksl] * a
                return carry2

            lax.fori_loop(0, _B // 16, _grp, 0)
            scs[sl_] = pltpu.async_copy(
                rows.at[sl_], acc.at[dstc.at[b]], ssem.at[sl_], add=True)
            if b + _NSLOT < _CHB:
                scs[sl_].wait()
                gats[sl_] = pltpu.async_copy(
                    xt_hbm.at[srcc.at[pl.ds((b + _NSLOT) * _B, _B)]],
                    rows.at[sl_], gsem.at[sl_])
        for h in scs:
            h.wait()
        return carry

    lax.fori_loop(0, _NCH, _chunk, 0)
    plsc.subcore_barrier()

    for k in range(_RPT // _ZR):
        rs = s * _RPT + k * _ZR
        pltpu.sync_copy(acc.at[pl.ds(rs, _ZR)], zbuf)
        pltpu.sync_copy(zbuf, out_hbm.at[c, pl.ds(rs, _ZR)])


def kernel(x, edge_index, edge_attr, W0, b0, Watt0, batt0, W1, b1, Watt1,
           batt1):
    src = edge_index[0]
    dst = edge_index[1].reshape(_NW, _NCH, _CHB, _B)
    ws0 = Watt0[:D, 0].reshape(1, D)
    wd0 = Watt0[D:2 * D, 0].reshape(1, D)
    we0 = Watt0[2 * D:, 0].reshape(DE, 1)
    ws1 = Watt1[:D, 0].reshape(1, D)
    wd1 = Watt1[D:2 * D, 0].reshape(1, D)
    we1 = Watt1[2 * D:, 0].reshape(DE, 1)

    ae0, ae1 = _edge_call(edge_attr.T, we0, we1,
                          batt0.reshape(1, 1), batt1.reshape(1, 1))
    ae0 = ae0.reshape(E)
    ae1 = ae1.reshape(E)

    xt0, as0, ad0 = _pre0_call(x, W0.T, b0.reshape(1, D), ws0, wd0)
    part = _sc_agg(xt0, src, dst, ae0, as0.reshape(N), ad0.reshape(N))
    xt1, as1, ad1 = _mid_call(part[0, :N], part[1, :N], W1.T,
                              b1.reshape(1, D), ws1, wd1)
    part = _sc_agg(xt1, src, dst, ae1, as1.reshape(N), ad1.reshape(N))
    return _post_call(part[0, :N], part[1, :N])


# speedup vs baseline: 12.0372x; 1.1717x over previous
import functools

import jax
import jax.numpy as jnp
from jax import lax
from jax.experimental import pallas as pl
from jax.experimental.pallas import tpu as pltpu
from jax.experimental.pallas import tpu_sc as plsc

N = 10000
E = 320000
D = 128
DE = 16


_MAXN = 1.0 - 1e-5


def _artanh(x):
    x = jnp.clip(x, -1.0 + 1e-7, 1.0 - 1e-7)
    return 0.5 * jnp.log((1.0 + x) / (1.0 - x))


def _rownorm(x):
    return jnp.sqrt(jnp.sum(x * x, axis=-1, keepdims=True))


def _proj(x):
    n = jnp.maximum(_rownorm(x), 1e-15)
    return jnp.where(n > _MAXN, x / n * _MAXN, x)


def _expmap0(u):
    n = jnp.maximum(_rownorm(u), 1e-15)
    return jnp.tanh(n) * u / n


def _logmap0(p):
    n = jnp.maximum(_rownorm(p), 1e-15)
    return _artanh(n) * p / n


def _dense_pre(xh, wt, b):
    xn = jnp.maximum(_rownorm(xh), 1e-15)
    mx = jnp.dot(xh, wt, preferred_element_type=jnp.float32)
    mxn = jnp.maximum(_rownorm(mx), 1e-15)
    mv = jnp.tanh(mxn / xn * _artanh(xn)) * mx / mxn
    mv = _proj(mv)
    hb = _proj(_expmap0(b))
    x2 = jnp.sum(mv * mv, axis=-1, keepdims=True)
    y2 = jnp.sum(hb * hb, axis=-1, keepdims=True)
    xy = jnp.sum(mv * hb, axis=-1, keepdims=True)
    num = (1.0 + 2.0 * xy + y2) * mv + (1.0 - x2) * hb
    den = 1.0 + 2.0 * xy + x2 * y2
    h = _proj(num / jnp.maximum(den, 1e-15))
    return _logmap0(h)


def _dense_post(s):
    h = _proj(_expmap0(s))
    xt = jnp.tanh(_logmap0(h))
    return _proj(_expmap0(xt))


_BLK = 2000
_EBLK = 6400


def _hyp_scale(n):
    t = jnp.tanh(n)
    f = jnp.where(t > _MAXN, _MAXN / n, t / n)
    nh = jnp.minimum(t, _MAXN)
    return _artanh(nh) / jnp.maximum(nh, 1e-15) * f


def _edge_body(eat_ref, we0_ref, we1_ref, b0_ref, b1_ref, ae0_ref, ae1_ref):
    x = eat_ref[...]
    q = jnp.sum(x * x, axis=0, keepdims=True)
    na = jnp.maximum(jnp.sqrt(q), 1e-15)
    p0 = jnp.sum(x * we0_ref[...], axis=0, keepdims=True)
    p1 = jnp.sum(x * we1_ref[...], axis=0, keepdims=True)
    s0 = _hyp_scale(na)
    ae0_ref[...] = (s0 * p0 + b0_ref[0, 0]).reshape(1, 1, _EBLK)
    n1 = jnp.maximum(s0 * na, 1e-15)
    s1 = _hyp_scale(n1)
    ae1_ref[...] = (s1 * s0 * p1 + b1_ref[0, 0]).reshape(1, 1, _EBLK)


def _edge_call(eat, we0, we1, b0, b1):
    grid = E // _EBLK
    colw = pl.BlockSpec((DE, 1), lambda i: (0, 0))
    one = pl.BlockSpec((1, 1), lambda i: (0, 0))
    return pl.pallas_call(
        _edge_body,
        grid=(grid,),
        in_specs=[pl.BlockSpec((DE, _EBLK), lambda i: (0, i)),
                  colw, colw, one, one],
        out_specs=[pl.BlockSpec((1, 1, _EBLK), lambda i: (i, 0, 0)),
                   pl.BlockSpec((1, 1, _EBLK), lambda i: (i, 0, 0))],
        out_shape=[jax.ShapeDtypeStruct((E // _EBLK, 1, _EBLK), jnp.float32),
                   jax.ShapeDtypeStruct((E // _EBLK, 1, _EBLK), jnp.float32)],
    )(eat, we0, we1, b0, b1)


def _pre_tail(xt, ws_ref, wd_ref, xt_ref, as_ref, ad_ref):
    xt_ref[...] = xt
    as_ref[...] = jnp.sum(xt * ws_ref[...], axis=1, keepdims=True)
    ad_ref[...] = jnp.sum(xt * wd_ref[...], axis=1, keepdims=True)


def _pre0_body(x_ref, wt_ref, b_ref, ws_ref, wd_ref, xt_ref, as_ref, ad_ref):
    xh = _proj(_expmap0(x_ref[...]))
    xt = _dense_pre(xh, wt_ref[...], b_ref[...])
    _pre_tail(xt, ws_ref, wd_ref, xt_ref, as_ref, ad_ref)


def _mid_body(sa_ref, sb_ref, wt_ref, b_ref, ws_ref, wd_ref,
              xt_ref, as_ref, ad_ref):
    xh = _dense_post(sa_ref[...] + sb_ref[...])
    xt = _dense_pre(xh, wt_ref[...], b_ref[...])
    _pre_tail(xt, ws_ref, wd_ref, xt_ref, as_ref, ad_ref)


def _post_body(sa_ref, sb_ref, o_ref):
    o_ref[...] = _dense_post(sa_ref[...] + sb_ref[...])


_rows = pl.BlockSpec((_BLK, D), lambda i: (i, 0))
_full = pl.BlockSpec((D, D), lambda i: (0, 0))
_vec = pl.BlockSpec((1, D), lambda i: (0, 0))
_col = pl.BlockSpec((_BLK, 1), lambda i: (i, 0))
_node_outs = [jax.ShapeDtypeStruct((N, D), jnp.float32),
              jax.ShapeDtypeStruct((N, 1), jnp.float32),
              jax.ShapeDtypeStruct((N, 1), jnp.float32)]


def _pre0_call(x, wt, b, ws, wd):
    return pl.pallas_call(
        _pre0_body, grid=(N // _BLK,),
        in_specs=[_rows, _full, _vec, _vec, _vec],
        out_specs=[_rows, _col, _col],
        out_shape=_node_outs,
    )(x, wt, b, ws, wd)


def _mid_call(sa, sb, wt, b, ws, wd):
    return pl.pallas_call(
        _mid_body, grid=(N // _BLK,),
        in_specs=[_rows, _rows, _full, _vec, _vec, _vec],
        out_specs=[_rows, _col, _col],
        out_shape=_node_outs,
    )(sa, sb, wt, b, ws, wd)


def _post_call(sa, sb):
    return pl.pallas_call(
        _post_body, grid=(N // _BLK,),
        in_specs=[_rows, _rows],
        out_specs=_rows,
        out_shape=jax.ShapeDtypeStruct((N, D), jnp.float32),
    )(sa, sb)


_NC = 2
_NS = 16
_NW = _NC * _NS
_B = 64
_CHB = 6
_NCH = 26
_EPT = _NCH * _CHB * _B
_TAIL = E - _NW * _EPT
_NP = 10240
_RPT = _NP // _NS
_NSLOT = 3

_sc_mesh = plsc.VectorSubcoreMesh(core_axis_name="c", subcore_axis_name="s")


@functools.partial(
    pl.kernel, mesh=_sc_mesh,
    out_type=jax.ShapeDtypeStruct((_NC, _NP, D), jnp.float32),
    compiler_params=pltpu.CompilerParams(needs_layout_passes=False),
    scratch_types=[
        pltpu.VMEM((N,), jnp.float32),
        pltpu.VMEM((N,), jnp.float32),
        pltpu.VMEM((_CHB * _B,), jnp.int32),
        pltpu.VMEM((_CHB * _B,), jnp.int32),
        pltpu.VMEM((_CHB, _B), jnp.int32),
        pltpu.VMEM((_CHB, _B), jnp.int32),
        pltpu.VMEM((_CHB * _B,), jnp.float32),
        pltpu.VMEM((_CHB * _B,), jnp.float32),
        pltpu.VMEM((_NSLOT, _B, D), jnp.float32),
        pltpu.VMEM_SHARED((_NP, D), jnp.float32),
        pltpu.SemaphoreType.DMA((_NSLOT,)),
        pltpu.SemaphoreType.DMA((_NSLOT,)),
        pltpu.SemaphoreType.DMA((2,)),
    ],
)
def _sc_agg(xt_hbm, src_hbm, dst_hbm, ae_hbm, asrc_hbm, adst_hbm, out_hbm,
            asrc_v, adst_v, srcc0, srcc1, dstc0, dstc1, aec0, aec1, rows,
            acc, gsem, ssem, isem):
    c = lax.axis_index("c")
    s = lax.axis_index("s")
    wid = c * _NS + s
    base = wid * _EPT
    srcc = (srcc0, srcc1)
    dstc = (dstc0, dstc1)
    aec = (aec0, aec1)

    def _zrow(i, carry):
        for k in range(D // 16):
            rows[0, i, pl.ds(k * 16, 16)] = jnp.zeros((16,), jnp.float32)
        return carry

    lax.fori_loop(0, _B, _zrow, 0)
    for k in range(_RPT // _B):
        pltpu.sync_copy(rows.at[0], acc.at[pl.ds(s * _RPT + k * _B, _B)])
    plsc.subcore_barrier()

    pltpu.sync_copy(asrc_hbm, asrc_v)
    pltpu.sync_copy(adst_hbm, adst_v)

    def _idx_dmas(ch, ph):
        off = base + ch * (_CHB * _B)
        ds = [pltpu.make_async_copy(src_hbm.at[pl.ds(off, _CHB * _B)],
                                    srcc[ph], isem.at[ph]),
              pltpu.make_async_copy(ae_hbm.at[pl.ds(off, _CHB * _B)],
                                    aec[ph], isem.at[ph])]
        for b in range(_CHB):
            ds.append(pltpu.make_async_copy(
                dst_hbm.at[pl.ds(off + b * _B, _B)], dstc[ph].at[b],
                isem.at[ph]))
        return ds

    def _prefetch(ch, ph):
        for d_ in _idx_dmas(ch, ph):
            d_.start()

    def _scale_block(srow, ph, b):
        def _grp(g, carry2):
            gsl = pl.ds(b * _B + g * 16, 16)
            a_s = plsc.load_gather(asrc_v, [srcc[ph][gsl]])
            a_d = plsc.load_gather(adst_v, [dstc[ph][b, pl.ds(g * 16, 16)]])
            att16 = 1.0 / (1.0 + jnp.exp(-(a_s + a_d + aec[ph][gsl])))
            for l in range(16):
                a = att16[l]
                i = g * 16 + l
                for k in range(D // 16):
                    ksl = pl.ds(k * 16, 16)
                    rows[srow, i, ksl] = rows[srow, i, ksl] * a
            return carry2

        lax.fori_loop(0, _B // 16, _grp, 0)

    def _gather(srow, ph, b):
        return pltpu.async_copy(
            xt_hbm.at[srcc[ph].at[pl.ds(b * _B, _B)]], rows.at[srow],
            gsem.at[srow])

    def _process_chunk(ch, ph, prefetch):
        for d_ in _idx_dmas(ch, ph):
            d_.wait()
        prefetch()
        gats = [_gather(b, ph, b) for b in range(_NSLOT)]
        scs = [None] * _NSLOT
        for b in range(_CHB):
            sl_ = b % _NSLOT
            if 2 <= b < _CHB - 1:
                t = (b + 1) % _NSLOT
                scs[t].wait()
                gats[t] = _gather(t, ph, b + 1)
            gats[sl_].wait()
            _scale_block(sl_, ph, b)
            scs[sl_] = pltpu.async_copy(
                rows.at[sl_], acc.at[dstc[ph].at[b]], ssem.at[sl_],
                add=True)
        for h in scs:
            h.wait()

    _prefetch(0, 0)

    def _two_chunks(it, carry):
        _process_chunk(2 * it, 0, lambda: _prefetch(2 * it + 1, 1))
        not_last = it != _NCH // 2 - 1

        def _guarded_prefetch():
            @pl.when(not_last)
            def _():
                _prefetch(2 * it + 2, 0)

        _process_chunk(2 * it + 1, 1, _guarded_prefetch)
        return carry

    lax.fori_loop(0, _NCH // 2, _two_chunks, 0)

    @pl.when(wid < _TAIL // _B)
    def _tail_block():
        off = _NW * _EPT + wid * _B
        pltpu.sync_copy(src_hbm.at[pl.ds(off, _B)],
                        srcc[0].at[pl.ds(0, _B)])
        pltpu.sync_copy(dst_hbm.at[pl.ds(off, _B)], dstc[0].at[0])
        pltpu.sync_copy(ae_hbm.at[pl.ds(off, _B)], aec[0].at[pl.ds(0, _B)])
        _gather(0, 0, 0).wait()
        _scale_block(0, 0, 0)
        pltpu.sync_copy(rows.at[0], acc.at[dstc[0].at[0]], add=True)

    plsc.subcore_barrier()

    for k in range(_RPT // _B):
        rs = s * _RPT + k * _B
        pltpu.sync_copy(acc.at[pl.ds(rs, _B)], rows.at[0])
        pltpu.sync_copy(rows.at[0], out_hbm.at[c, pl.ds(rs, _B)])


def kernel(x, edge_index, edge_attr, W0, b0, Watt0, batt0, W1, b1, Watt1,
           batt1):
    src = edge_index[0]
    dst = edge_index[1]
    ws0 = Watt0[:D, 0].reshape(1, D)
    wd0 = Watt0[D:2 * D, 0].reshape(1, D)
    we0 = Watt0[2 * D:, 0].reshape(DE, 1)
    ws1 = Watt1[:D, 0].reshape(1, D)
    wd1 = Watt1[D:2 * D, 0].reshape(1, D)
    we1 = Watt1[2 * D:, 0].reshape(DE, 1)

    ae0, ae1 = _edge_call(edge_attr.T, we0, we1,
                          batt0.reshape(1, 1), batt1.reshape(1, 1))
    ae0 = ae0.reshape(E)
    ae1 = ae1.reshape(E)

    xt0, as0, ad0 = _pre0_call(x, W0.T, b0.reshape(1, D), ws0, wd0)
    part = _sc_agg(xt0, src, dst, ae0, as0.reshape(N), ad0.reshape(N))
    xt1, as1, ad1 = _mid_call(part[0, :N], part[1, :N], W1.T,
                              b1.reshape(1, D), ws1, wd1)
    part = _sc_agg(xt1, src, dst, ae1, as1.reshape(N), ad1.reshape(N))
    return _post_call(part[0, :N], part[1, :N])


# speedup vs baseline: 12.5135x; 1.0396x over previous
import functools

import jax
import jax.numpy as jnp
from jax import lax
from jax.experimental import pallas as pl
from jax.experimental.pallas import tpu as pltpu
from jax.experimental.pallas import tpu_sc as plsc

N = 10000
E = 320000
D = 128
DE = 16


_MAXN = 1.0 - 1e-5


def _artanh(x):
    x = jnp.clip(x, -1.0 + 1e-7, 1.0 - 1e-7)
    return 0.5 * jnp.log((1.0 + x) / (1.0 - x))


def _rownorm(x):
    return jnp.sqrt(jnp.sum(x * x, axis=-1, keepdims=True))


def _proj(x):
    n = jnp.maximum(_rownorm(x), 1e-15)
    return jnp.where(n > _MAXN, x / n * _MAXN, x)


def _expmap0(u):
    n = jnp.maximum(_rownorm(u), 1e-15)
    return jnp.tanh(n) * u / n


def _logmap0(p):
    n = jnp.maximum(_rownorm(p), 1e-15)
    return _artanh(n) * p / n


def _dense_pre(xh, wt, b):
    xn = jnp.maximum(_rownorm(xh), 1e-15)
    mx = jnp.dot(xh, wt, preferred_element_type=jnp.float32)
    mxn = jnp.maximum(_rownorm(mx), 1e-15)
    mv = jnp.tanh(mxn / xn * _artanh(xn)) * mx / mxn
    mv = _proj(mv)
    hb = _proj(_expmap0(b))
    x2 = jnp.sum(mv * mv, axis=-1, keepdims=True)
    y2 = jnp.sum(hb * hb, axis=-1, keepdims=True)
    xy = jnp.sum(mv * hb, axis=-1, keepdims=True)
    num = (1.0 + 2.0 * xy + y2) * mv + (1.0 - x2) * hb
    den = 1.0 + 2.0 * xy + x2 * y2
    h = _proj(num / jnp.maximum(den, 1e-15))
    return _logmap0(h)


def _dense_post(s):
    h = _proj(_expmap0(s))
    xt = jnp.tanh(_logmap0(h))
    return _proj(_expmap0(xt))


_BLK = 2000
_EBLK = 6400


def _hyp_scale(n):
    t = jnp.tanh(n)
    f = jnp.where(t > _MAXN, _MAXN / n, t / n)
    nh = jnp.minimum(t, _MAXN)
    return _artanh(nh) / jnp.maximum(nh, 1e-15) * f


def _edge_body(layer, eat_ref, we_ref, b_ref, ae_ref):
    x = eat_ref[...]
    q = jnp.sum(x * x, axis=0, keepdims=True)
    na = jnp.maximum(jnp.sqrt(q), 1e-15)
    p = jnp.sum(x * we_ref[...], axis=0, keepdims=True)
    s0 = _hyp_scale(na)
    s = s0
    if layer == 1:
        n1 = jnp.maximum(s0 * na, 1e-15)
        s = _hyp_scale(n1) * s0
    ae_ref[...] = (s * p + b_ref[0, 0]).reshape(1, 1, _EBLK)


def _edge_call(layer, eat, we, b):
    grid = E // _EBLK
    colw = pl.BlockSpec((DE, 1), lambda i: (0, 0))
    one = pl.BlockSpec((1, 1), lambda i: (0, 0))
    return pl.pallas_call(
        functools.partial(_edge_body, layer),
        grid=(grid,),
        in_specs=[pl.BlockSpec((DE, _EBLK), lambda i: (0, i)),
                  colw, one],
        out_specs=pl.BlockSpec((1, 1, _EBLK), lambda i: (i, 0, 0)),
        out_shape=jax.ShapeDtypeStruct((E // _EBLK, 1, _EBLK), jnp.float32),
    )(eat, we, b)


def _pre_tail(xt, ws_ref, wd_ref, xt_ref, as_ref, ad_ref):
    xt_ref[...] = xt
    as_ref[...] = jnp.sum(xt * ws_ref[...], axis=1, keepdims=True)
    ad_ref[...] = jnp.sum(xt * wd_ref[...], axis=1, keepdims=True)


def _pre0_body(x_ref, wt_ref, b_ref, ws_ref, wd_ref, xt_ref, as_ref, ad_ref):
    xh = _proj(_expmap0(x_ref[...]))
    xt = _dense_pre(xh, wt_ref[...], b_ref[...])
    _pre_tail(xt, ws_ref, wd_ref, xt_ref, as_ref, ad_ref)


def _mid_body(sa_ref, sb_ref, wt_ref, b_ref, ws_ref, wd_ref,
              xt_ref, as_ref, ad_ref):
    xh = _dense_post(sa_ref[0] + sb_ref[0])
    xt = _dense_pre(xh, wt_ref[...], b_ref[...])
    _pre_tail(xt, ws_ref, wd_ref, xt_ref, as_ref, ad_ref)


def _post_body(sa_ref, sb_ref, o_ref):
    o_ref[...] = _dense_post(sa_ref[0] + sb_ref[0])


_rows = pl.BlockSpec((_BLK, D), lambda i: (i, 0))
_full = pl.BlockSpec((D, D), lambda i: (0, 0))
_vec = pl.BlockSpec((1, D), lambda i: (0, 0))
_col = pl.BlockSpec((_BLK, 1), lambda i: (i, 0))
_node_outs = [jax.ShapeDtypeStruct((N, D), jnp.float32),
              jax.ShapeDtypeStruct((N, 1), jnp.float32),
              jax.ShapeDtypeStruct((N, 1), jnp.float32)]


def _pre0_call(x, wt, b, ws, wd):
    return pl.pallas_call(
        _pre0_body, grid=(N // _BLK,),
        in_specs=[_rows, _full, _vec, _vec, _vec],
        out_specs=[_rows, _col, _col],
        out_shape=_node_outs,
    )(x, wt, b, ws, wd)


_part_a = pl.BlockSpec((1, _BLK, D), lambda i: (0, i, 0))
_part_b = pl.BlockSpec((1, _BLK, D), lambda i: (1, i, 0))


def _mid_call(part, wt, b, ws, wd):
    return pl.pallas_call(
        _mid_body, grid=(N // _BLK,),
        in_specs=[_part_a, _part_b, _full, _vec, _vec, _vec],
        out_specs=[_rows, _col, _col],
        out_shape=_node_outs,
    )(part, part, wt, b, ws, wd)


def _post_call(part):
    return pl.pallas_call(
        _post_body, grid=(N // _BLK,),
        in_specs=[_part_a, _part_b],
        out_specs=_rows,
        out_shape=jax.ShapeDtypeStruct((N, D), jnp.float32),
    )(part, part)


_NC = 2
_NS = 16
_NW = _NC * _NS
_B = 64
_CHB = 6
_NCH = 26
_EPT = _NCH * _CHB * _B
_TAIL = E - _NW * _EPT
_NP = 10240
_RPT = _NP // _NS
_NSLOT = 3

_sc_mesh = plsc.VectorSubcoreMesh(core_axis_name="c", subcore_axis_name="s")


@functools.partial(
    pl.kernel, mesh=_sc_mesh,
    out_type=jax.ShapeDtypeStruct((_NC, _NP, D), jnp.float32),
    compiler_params=pltpu.CompilerParams(needs_layout_passes=False),
    scratch_types=[
        pltpu.VMEM((N,), jnp.float32),
        pltpu.VMEM((N,), jnp.float32),
        pltpu.VMEM((_CHB * _B,), jnp.int32),
        pltpu.VMEM((_CHB * _B,), jnp.int32),
        pltpu.VMEM((_CHB, _B), jnp.int32),
        pltpu.VMEM((_CHB, _B), jnp.int32),
        pltpu.VMEM((_CHB * _B,), jnp.float32),
        pltpu.VMEM((_CHB * _B,), jnp.float32),
        pltpu.VMEM((_NSLOT, _B, D), jnp.float32),
        pltpu.VMEM_SHARED((_NP, D), jnp.float32),
        pltpu.SemaphoreType.DMA((_NSLOT,)),
        pltpu.SemaphoreType.DMA((_NSLOT,)),
        pltpu.SemaphoreType.DMA((2,)),
    ],
)
def _sc_agg(xt_hbm, src_hbm, dst_hbm, ae_hbm, asrc_hbm, adst_hbm, out_hbm,
            asrc_v, adst_v, srcc0, srcc1, dstc0, dstc1, aec0, aec1, rows,
            acc, gsem, ssem, isem):
    c = lax.axis_index("c")
    s = lax.axis_index("s")
    wid = c * _NS + s
    base = wid * _EPT
    srcc = (srcc0, srcc1)
    dstc = (dstc0, dstc1)
    aec = (aec0, aec1)

    def _zrow(i, carry):
        for k in range(D // 16):
            rows[0, i, pl.ds(k * 16, 16)] = jnp.zeros((16,), jnp.float32)
        return carry

    lax.fori_loop(0, _B, _zrow, 0)
    for k in range(_RPT // _B):
        pltpu.sync_copy(rows.at[0], acc.at[pl.ds(s * _RPT + k * _B, _B)])
    plsc.subcore_barrier()

    pltpu.sync_copy(asrc_hbm, asrc_v)
    pltpu.sync_copy(adst_hbm, adst_v)

    def _idx_dmas(ch, ph):
        off = base + ch * (_CHB * _B)
        ds = [pltpu.make_async_copy(src_hbm.at[pl.ds(off, _CHB * _B)],
                                    srcc[ph], isem.at[ph]),
              pltpu.make_async_copy(ae_hbm.at[pl.ds(off, _CHB * _B)],
                                    aec[ph], isem.at[ph])]
        for b in range(_CHB):
            ds.append(pltpu.make_async_copy(
                dst_hbm.at[pl.ds(off + b * _B, _B)], dstc[ph].at[b],
                isem.at[ph]))
        return ds

    def _prefetch(ch, ph):
        for d_ in _idx_dmas(ch, ph):
            d_.start()

    def _scale_block(srow, ph, b):
        def _grp(g, carry2):
            gsl = pl.ds(b * _B + g * 16, 16)
            a_s = plsc.load_gather(asrc_v, [srcc[ph][gsl]])
            a_d = plsc.load_gather(adst_v, [dstc[ph][b, pl.ds(g * 16, 16)]])
            att16 = 1.0 / (1.0 + jnp.exp(-(a_s + a_d + aec[ph][gsl])))
            for l in range(16):
                a = att16[l]
                i = g * 16 + l
                for k in range(D // 16):
                    ksl = pl.ds(k * 16, 16)
                    rows[srow, i, ksl] = rows[srow, i, ksl] * a
            return carry2

        lax.fori_loop(0, _B // 16, _grp, 0)

    def _gather(srow, ph, b):
        return pltpu.async_copy(
            xt_hbm.at[srcc[ph].at[pl.ds(b * _B, _B)]], rows.at[srow],
            gsem.at[srow])

    def _drain_tail_scatters(ph):
        for s_ in range(_NSLOT):
            pltpu.make_async_copy(
                rows.at[s_], acc.at[dstc[ph].at[_NSLOT + s_]],
                ssem.at[s_]).wait()

    def _process_chunk(ch, ph, prefetch, drain_prev):
        for d_ in _idx_dmas(ch, ph):
            d_.wait()
        drain_prev()
        prefetch()
        gats = [_gather(b, ph, b) for b in range(_NSLOT)]
        scs = [None] * _NSLOT
        for b in range(_CHB):
            sl_ = b % _NSLOT
            if 2 <= b < _CHB - 1:
                t = (b + 1) % _NSLOT
                scs[t].wait()
                gats[t] = _gather(t, ph, b + 1)
            gats[sl_].wait()
            _scale_block(sl_, ph, b)
            scs[sl_] = pltpu.async_copy(
                rows.at[sl_], acc.at[dstc[ph].at[b]], ssem.at[sl_],
                add=True)

    _prefetch(0, 0)

    def _two_chunks(it, carry):
        def _drain_first():
            @pl.when(it != 0)
            def _():
                _drain_tail_scatters(1)

        _process_chunk(2 * it, 0, lambda: _prefetch(2 * it + 1, 1),
                       _drain_first)
        not_last = it != _NCH // 2 - 1

        def _guarded_prefetch():
            @pl.when(not_last)
            def _():
                _prefetch(2 * it + 2, 0)

        _process_chunk(2 * it + 1, 1, _guarded_prefetch,
                       lambda: _drain_tail_scatters(0))
        return carry

    lax.fori_loop(0, _NCH // 2, _two_chunks, 0)
    _drain_tail_scatters(1)

    @pl.when(wid < _TAIL // _B)
    def _tail_block():
        off = _NW * _EPT + wid * _B
        pltpu.sync_copy(src_hbm.at[pl.ds(off, _B)],
                        srcc[0].at[pl.ds(0, _B)])
        pltpu.sync_copy(dst_hbm.at[pl.ds(off, _B)], dstc[0].at[0])
        pltpu.sync_copy(ae_hbm.at[pl.ds(off, _B)], aec[0].at[pl.ds(0, _B)])
        _gather(0, 0, 0).wait()
        _scale_block(0, 0, 0)
        pltpu.sync_copy(rows.at[0], acc.at[dstc[0].at[0]], add=True)

    plsc.subcore_barrier()

    for k in range(_RPT // _B):
        rs = s * _RPT + k * _B
        pltpu.sync_copy(acc.at[pl.ds(rs, _B)], rows.at[0])
        pltpu.sync_copy(rows.at[0], out_hbm.at[c, pl.ds(rs, _B)])


def kernel(x, edge_index, edge_attr, W0, b0, Watt0, batt0, W1, b1, Watt1,
           batt1):
    src = edge_index[0]
    dst = edge_index[1]
    ws0 = Watt0[:D, 0].reshape(1, D)
    wd0 = Watt0[D:2 * D, 0].reshape(1, D)
    we0 = Watt0[2 * D:, 0].reshape(DE, 1)
    ws1 = Watt1[:D, 0].reshape(1, D)
    wd1 = Watt1[D:2 * D, 0].reshape(1, D)
    we1 = Watt1[2 * D:, 0].reshape(DE, 1)

    eat = edge_attr.T
    ae0 = _edge_call(0, eat, we0, batt0.reshape(1, 1)).reshape(E)
    ae1 = _edge_call(1, eat, we1, batt1.reshape(1, 1)).reshape(E)

    xt0, as0, ad0 = _pre0_call(x, W0.T, b0.reshape(1, D), ws0, wd0)
    part = _sc_agg(xt0, src, dst, ae0, as0.reshape(N), ad0.reshape(N))
    xt1, as1, ad1 = _mid_call(part, W1.T, b1.reshape(1, D), ws1, wd1)
    part = _sc_agg(xt1, src, dst, ae1, as1.reshape(N), ad1.reshape(N))
    return _post_call(part)


# speedup vs baseline: 12.9069x; 1.0314x over previous
import functools

import jax
import jax.numpy as jnp
from jax import lax
from jax.experimental import pallas as pl
from jax.experimental.pallas import tpu as pltpu
from jax.experimental.pallas import tpu_sc as plsc

N = 10000
E = 320000
D = 128
DE = 16


_MAXN = 1.0 - 1e-5


def _artanh(x):
    x = jnp.clip(x, -1.0 + 1e-7, 1.0 - 1e-7)
    return 0.5 * jnp.log((1.0 + x) / (1.0 - x))


def _rownorm(x):
    return jnp.sqrt(jnp.sum(x * x, axis=-1, keepdims=True))


def _proj(x):
    n = jnp.maximum(_rownorm(x), 1e-15)
    return jnp.where(n > _MAXN, x / n * _MAXN, x)


def _expmap0(u):
    n = jnp.maximum(_rownorm(u), 1e-15)
    return jnp.tanh(n) * u / n


def _logmap0(p):
    n = jnp.maximum(_rownorm(p), 1e-15)
    return _artanh(n) * p / n


def _dense_pre(xh, wt, b):
    xn = jnp.maximum(_rownorm(xh), 1e-15)
    mx = jnp.dot(xh, wt, preferred_element_type=jnp.float32)
    mxn = jnp.maximum(_rownorm(mx), 1e-15)
    mv = jnp.tanh(mxn / xn * _artanh(xn)) * mx / mxn
    mv = _proj(mv)
    hb = _proj(_expmap0(b))
    x2 = jnp.sum(mv * mv, axis=-1, keepdims=True)
    y2 = jnp.sum(hb * hb, axis=-1, keepdims=True)
    xy = jnp.sum(mv * hb, axis=-1, keepdims=True)
    num = (1.0 + 2.0 * xy + y2) * mv + (1.0 - x2) * hb
    den = 1.0 + 2.0 * xy + x2 * y2
    h = _proj(num / jnp.maximum(den, 1e-15))
    return _logmap0(h)


def _dense_post(s):
    h = _proj(_expmap0(s))
    xt = jnp.tanh(_logmap0(h))
    return _proj(_expmap0(xt))


_BLK = 2000
_EBLK = 6400


def _hyp_scale(n):
    t = jnp.tanh(n)
    f = jnp.where(t > _MAXN, _MAXN / n, t / n)
    nh = jnp.minimum(t, _MAXN)
    return _artanh(nh) / jnp.maximum(nh, 1e-15) * f


def _edge_body(layer, eat_ref, we_ref, b_ref, ae_ref):
    x = eat_ref[...]
    q = jnp.sum(x * x, axis=0, keepdims=True)
    na = jnp.maximum(jnp.sqrt(q), 1e-15)
    p = jnp.sum(x * we_ref[...], axis=0, keepdims=True)
    s0 = _hyp_scale(na)
    s = s0
    if layer == 1:
        n1 = jnp.maximum(s0 * na, 1e-15)
        s = _hyp_scale(n1) * s0
    ae_ref[...] = (s * p + b_ref[0, 0]).reshape(1, 1, _EBLK)


def _edge_call(layer, eat, we, b):
    grid = E // _EBLK
    colw = pl.BlockSpec((DE, 1), lambda i: (0, 0))
    one = pl.BlockSpec((1, 1), lambda i: (0, 0))
    return pl.pallas_call(
        functools.partial(_edge_body, layer),
        grid=(grid,),
        in_specs=[pl.BlockSpec((DE, _EBLK), lambda i: (0, i)),
                  colw, one],
        out_specs=pl.BlockSpec((1, 1, _EBLK), lambda i: (i, 0, 0)),
        out_shape=jax.ShapeDtypeStruct((E // _EBLK, 1, _EBLK), jnp.float32),
    )(eat, we, b)


def _pre_tail(xt, ws_ref, wd_ref, xt_ref, as_ref, ad_ref):
    xt_ref[...] = xt
    as_ref[...] = jnp.sum(xt * ws_ref[...], axis=1, keepdims=True)
    ad_ref[...] = jnp.sum(xt * wd_ref[...], axis=1, keepdims=True)


def _pre0_body(x_ref, wt_ref, b_ref, ws_ref, wd_ref, xt_ref, as_ref, ad_ref):
    xh = _proj(_expmap0(x_ref[...]))
    xt = _dense_pre(xh, wt_ref[...], b_ref[...])
    _pre_tail(xt, ws_ref, wd_ref, xt_ref, as_ref, ad_ref)


def _mid_body(sa_ref, sb_ref, wt_ref, b_ref, ws_ref, wd_ref,
              xt_ref, as_ref, ad_ref):
    xh = _dense_post(sa_ref[0] + sb_ref[0])
    xt = _dense_pre(xh, wt_ref[...], b_ref[...])
    _pre_tail(xt, ws_ref, wd_ref, xt_ref, as_ref, ad_ref)


def _post_body(sa_ref, sb_ref, o_ref):
    o_ref[...] = _dense_post(sa_ref[0] + sb_ref[0])


_rows = pl.BlockSpec((_BLK, D), lambda i: (i, 0))
_full = pl.BlockSpec((D, D), lambda i: (0, 0))
_vec = pl.BlockSpec((1, D), lambda i: (0, 0))
_col = pl.BlockSpec((_BLK, 1), lambda i: (i, 0))
_node_outs = [jax.ShapeDtypeStruct((N, D), jnp.float32),
              jax.ShapeDtypeStruct((N, 1), jnp.float32),
              jax.ShapeDtypeStruct((N, 1), jnp.float32)]


def _pre0_call(x, wt, b, ws, wd):
    return pl.pallas_call(
        _pre0_body, grid=(N // _BLK,),
        in_specs=[_rows, _full, _vec, _vec, _vec],
        out_specs=[_rows, _col, _col],
        out_shape=_node_outs,
    )(x, wt, b, ws, wd)


_part_a = pl.BlockSpec((1, _BLK, D), lambda i: (0, i, 0))
_part_b = pl.BlockSpec((1, _BLK, D), lambda i: (1, i, 0))


def _mid_call(part, wt, b, ws, wd):
    return pl.pallas_call(
        _mid_body, grid=(N // _BLK,),
        in_specs=[_part_a, _part_b, _full, _vec, _vec, _vec],
        out_specs=[_rows, _col, _col],
        out_shape=_node_outs,
    )(part, part, wt, b, ws, wd)


def _post_call(part):
    return pl.pallas_call(
        _post_body, grid=(N // _BLK,),
        in_specs=[_part_a, _part_b],
        out_specs=_rows,
        out_shape=jax.ShapeDtypeStruct((N, D), jnp.float32),
    )(part, part)


_NC = 2
_NS = 16
_NW = _NC * _NS
_B = 64
_CHB = 6
_NCH = 26
_EPT = _NCH * _CHB * _B
_TAIL = E - _NW * _EPT
_NP = 10240
_RPT = _NP // _NS
_NSLOT = 3

_sc_mesh = plsc.VectorSubcoreMesh(core_axis_name="c", subcore_axis_name="s")


@functools.partial(
    pl.kernel, mesh=_sc_mesh,
    out_type=jax.ShapeDtypeStruct((_NC, _NP, D), jnp.float32),
    compiler_params=pltpu.CompilerParams(needs_layout_passes=False),
    scratch_types=[
        pltpu.VMEM((N,), jnp.float32),
        pltpu.VMEM((N,), jnp.float32),
        pltpu.VMEM((_CHB * _B,), jnp.int32),
        pltpu.VMEM((_CHB * _B,), jnp.int32),
        pltpu.VMEM((_CHB, _B), jnp.int32),
        pltpu.VMEM((_CHB, _B), jnp.int32),
        pltpu.VMEM((_CHB * _B,), jnp.float32),
        pltpu.VMEM((_CHB * _B,), jnp.float32),
        pltpu.VMEM((_NSLOT, _B, D), jnp.float32),
        pltpu.VMEM_SHARED((_NP, D), jnp.float32),
        pltpu.SemaphoreType.DMA((_NSLOT,)),
        pltpu.SemaphoreType.DMA((_NSLOT,)),
        pltpu.SemaphoreType.DMA((2,)),
    ],
)
def _sc_agg(xt_hbm, ei_hbm, ae_hbm, asrc_hbm, adst_hbm, out_hbm,
            asrc_v, adst_v, srcc0, srcc1, dstc0, dstc1, aec0, aec1, rows,
            acc, gsem, ssem, isem):
    c = lax.axis_index("c")
    s = lax.axis_index("s")
    wid = c * _NS + s
    base = wid * _EPT
    srcc = (srcc0, srcc1)
    dstc = (dstc0, dstc1)
    aec = (aec0, aec1)

    def _zrow(i, carry):
        for k in range(D // 16):
            rows[0, i, pl.ds(k * 16, 16)] = jnp.zeros((16,), jnp.float32)
        return carry

    lax.fori_loop(0, _B, _zrow, 0)
    for k in range(_RPT // _B):
        pltpu.sync_copy(rows.at[0], acc.at[pl.ds(s * _RPT + k * _B, _B)])
    plsc.subcore_barrier()

    pltpu.sync_copy(asrc_hbm, asrc_v)
    pltpu.sync_copy(adst_hbm, adst_v)

    def _idx_dmas(ch, ph):
        off = base + ch * (_CHB * _B)
        ds = [pltpu.make_async_copy(ei_hbm.at[0, pl.ds(off, _CHB * _B)],
                                    srcc[ph], isem.at[ph]),
              pltpu.make_async_copy(ae_hbm.at[pl.ds(off, _CHB * _B)],
                                    aec[ph], isem.at[ph])]
        for b in range(_CHB):
            ds.append(pltpu.make_async_copy(
                ei_hbm.at[1, pl.ds(off + b * _B, _B)], dstc[ph].at[b],
                isem.at[ph]))
        return ds

    def _prefetch(ch, ph):
        for d_ in _idx_dmas(ch, ph):
            d_.start()

    def _scale_block(srow, ph, b):
        def _grp(g, carry2):
            gsl = pl.ds(b * _B + g * 16, 16)
            a_s = plsc.load_gather(asrc_v, [srcc[ph][gsl]])
            a_d = plsc.load_gather(adst_v, [dstc[ph][b, pl.ds(g * 16, 16)]])
            att16 = 1.0 / (1.0 + jnp.exp(-(a_s + a_d + aec[ph][gsl])))
            for l in range(16):
                a = att16[l]
                i = g * 16 + l
                for k in range(D // 16):
                    ksl = pl.ds(k * 16, 16)
                    rows[srow, i, ksl] = rows[srow, i, ksl] * a
            return carry2

        lax.fori_loop(0, _B // 16, _grp, 0)

    def _gather(srow, ph, b):
        return pltpu.async_copy(
            xt_hbm.at[srcc[ph].at[pl.ds(b * _B, _B)]], rows.at[srow],
            gsem.at[srow])

    def _drain_tail_scatters(ph):
        for s_ in range(_NSLOT):
            pltpu.make_async_copy(
                rows.at[s_], acc.at[dstc[ph].at[_NSLOT + s_]],
                ssem.at[s_]).wait()

    def _process_chunk(ch, ph, prefetch, drain_prev):
        for d_ in _idx_dmas(ch, ph):
            d_.wait()
        drain_prev()
        prefetch()
        gats = [_gather(b, ph, b) for b in range(_NSLOT)]
        scs = [None] * _NSLOT
        for b in range(_CHB):
            sl_ = b % _NSLOT
            if 2 <= b < _CHB - 1:
                t = (b + 1) % _NSLOT
                scs[t].wait()
                gats[t] = _gather(t, ph, b + 1)
            gats[sl_].wait()
            _scale_block(sl_, ph, b)
            scs[sl_] = pltpu.async_copy(
                rows.at[sl_], acc.at[dstc[ph].at[b]], ssem.at[sl_],
                add=True)

    _prefetch(0, 0)

    def _two_chunks(it, carry):
        def _drain_first():
            @pl.when(it != 0)
            def _():
                _drain_tail_scatters(1)

        _process_chunk(2 * it, 0, lambda: _prefetch(2 * it + 1, 1),
                       _drain_first)
        not_last = it != _NCH // 2 - 1

        def _guarded_prefetch():
            @pl.when(not_last)
            def _():
                _prefetch(2 * it + 2, 0)

        _process_chunk(2 * it + 1, 1, _guarded_prefetch,
                       lambda: _drain_tail_scatters(0))
        return carry

    lax.fori_loop(0, _NCH // 2, _two_chunks, 0)
    _drain_tail_scatters(1)

    @pl.when(wid < _TAIL // _B)
    def _tail_block():
        off = _NW * _EPT + wid * _B
        pltpu.sync_copy(ei_hbm.at[0, pl.ds(off, _B)],
                        srcc[0].at[pl.ds(0, _B)])
        pltpu.sync_copy(ei_hbm.at[1, pl.ds(off, _B)], dstc[0].at[0])
        pltpu.sync_copy(ae_hbm.at[pl.ds(off, _B)], aec[0].at[pl.ds(0, _B)])
        _gather(0, 0, 0).wait()
        _scale_block(0, 0, 0)
        pltpu.sync_copy(rows.at[0], acc.at[dstc[0].at[0]], add=True)

    plsc.subcore_barrier()

    for k in range(_RPT // _B):
        rs = s * _RPT + k * _B
        pltpu.sync_copy(acc.at[pl.ds(rs, _B)], rows.at[0])
        pltpu.sync_copy(rows.at[0], out_hbm.at[c, pl.ds(rs, _B)])


def kernel(x, edge_index, edge_attr, W0, b0, Watt0, batt0, W1, b1, Watt1,
           batt1):
    ws0 = Watt0[:D, 0].reshape(1, D)
    wd0 = Watt0[D:2 * D, 0].reshape(1, D)
    we0 = Watt0[2 * D:, 0].reshape(DE, 1)
    ws1 = Watt1[:D, 0].reshape(1, D)
    wd1 = Watt1[D:2 * D, 0].reshape(1, D)
    we1 = Watt1[2 * D:, 0].reshape(DE, 1)

    eat = edge_attr.T
    ae0 = _edge_call(0, eat, we0, batt0.reshape(1, 1)).reshape(E)
    ae1 = _edge_call(1, eat, we1, batt1.reshape(1, 1)).reshape(E)

    xt0, as0, ad0 = _pre0_call(x, W0.T, b0.reshape(1, D), ws0, wd0)
    part = _sc_agg(xt0, edge_index, ae0, as0.reshape(N), ad0.reshape(N))
    xt1, as1, ad1 = _mid_call(part, W1.T, b1.reshape(1, D), ws1, wd1)
    part = _sc_agg(xt1, edge_index, ae1, as1.reshape(N), ad1.reshape(N))
    return _post_call(part)
